# baseline (device time: 129489 ns/iter reference)
import functools

import jax
import jax.numpy as jnp
from jax import lax
from jax.experimental import pallas as pl
from jax.experimental.pallas import tpu as pltpu

N_DEV = 4
SCALE = 0.08838834764831843
DH = 128
LOCAL_WINDOW = 128
GLOBAL_K = 32


def _allreduce_body(p_ref, out_ref, comm_ref, send_sems, recv_sems):
    my = lax.axis_index("i")
    left = (my - 1) % N_DEV
    right = (my + 1) % N_DEV

    barrier_sem = pltpu.get_barrier_semaphore()
    for nbr in [left, right]:
        pl.semaphore_signal(
            barrier_sem, inc=1,
            device_id=(nbr,), device_id_type=pl.DeviceIdType.MESH,
        )
    pl.semaphore_wait(barrier_sem, 2)

    comm_ref[0] = p_ref[...]
    out_ref[...] = p_ref[...].astype(jnp.float32)

    for h in range(N_DEV - 1):
        rdma = pltpu.make_async_remote_copy(
            src_ref=comm_ref.at[h],
            dst_ref=comm_ref.at[h + 1],
            send_sem=send_sems.at[h],
            recv_sem=recv_sems.at[h + 1],
            device_id=(right,),
            device_id_type=pl.DeviceIdType.MESH,
        )
        rdma.start()
        rdma.wait()
        out_ref[...] += comm_ref[h + 1].astype(jnp.float32)

    @functools.partial(pl.run_scoped, sem=pltpu.SemaphoreType.REGULAR)
    def _(sem):
        for nbr in [left, right]:
            pl.semaphore_signal(
                sem, inc=1,
                device_id=(nbr,), device_id_type=pl.DeviceIdType.MESH,
            )
        pl.semaphore_wait(sem, 2)


def _ring_allreduce(partial):
    s, d = partial.shape
    return pl.pallas_call(
        _allreduce_body,
        out_shape=jax.ShapeDtypeStruct((s, d), jnp.float32),
        in_specs=[pl.BlockSpec(memory_space=pltpu.VMEM)],
        out_specs=pl.BlockSpec(memory_space=pltpu.VMEM),
        scratch_shapes=[
            pltpu.VMEM((N_DEV, s, d), partial.dtype),
            pltpu.SemaphoreType.DMA((N_DEV,)),
            pltpu.SemaphoreType.DMA((N_DEV,)),
        ],
        compiler_params=pltpu.CompilerParams(collective_id=0),
    )(partial)


def kernel(x, Wq, K_ext, V_ext, Wo):
    i = lax.axis_index("i")
    sq = x.shape[1]
    skv = K_ext.shape[1]
    hq_local = Wq.shape[1] // DH

    xb = x[0].astype(jnp.bfloat16)
    q = (xb @ Wq.astype(jnp.bfloat16)).reshape(sq, hq_local, DH)
    k = lax.dynamic_slice_in_dim(K_ext[0], i * hq_local, hq_local, axis=1)
    v = lax.dynamic_slice_in_dim(V_ext[0], i * hq_local, hq_local, axis=1)

    scores = jnp.einsum(
        "ihd,jhd->hij", q, k.astype(jnp.bfloat16),
        preferred_element_type=jnp.float32,
    ) * SCALE
    qi = lax.broadcasted_iota(jnp.int32, (sq, skv), 0)
    ki = lax.broadcasted_iota(jnp.int32, (sq, skv), 1)
    mask = (jnp.abs(qi - ki) <= LOCAL_WINDOW) | (ki < GLOBAL_K) | (qi < GLOBAL_K)
    scores = jnp.where(mask[None, :, :], scores, -1e9)
    w = jax.nn.softmax(scores, axis=-1)

    ctx = jnp.einsum(
        "hij,jhd->ihd", w.astype(jnp.bfloat16), v.astype(jnp.bfloat16),
        preferred_element_type=jnp.float32,
    ).reshape(sq, hq_local * DH)
    partial = (ctx.astype(jnp.bfloat16) @ Wo.astype(jnp.bfloat16)).astype(
        jnp.bfloat16
    )

    out = _ring_allreduce(partial)
    return out.reshape(1, sq, Wo.shape[1])


# device time: 96773 ns/iter; 1.3381x vs baseline; 1.3381x over previous
import functools

import jax
import jax.numpy as jnp
from jax import lax
from jax.experimental import pallas as pl
from jax.experimental.pallas import tpu as pltpu

N_DEV = 4
SCALE = 0.08838834764831843
DH = 128
LOCAL_WINDOW = 128
GLOBAL_K = 32


def _allreduce_body(
    p_ref, out_ref, r1, s2, r2, s3, r3, s4, r4, acch, accq,
    send_sems, recv_sems,
):
    my = lax.axis_index("i")
    p1 = my ^ 1
    p2 = 3 - my

    barrier_sem = pltpu.get_barrier_semaphore()
    for nbr in [p1, p2]:
        pl.semaphore_signal(
            barrier_sem, inc=1,
            device_id=(nbr,), device_id_type=pl.DeviceIdType.MESH,
        )
    pl.semaphore_wait(barrier_sem, 2)

    sq, d = out_ref.shape
    h, q = sq // 2, sq // 4
    keep_top = (my == 0) | (my == 3)
    keep_off = jnp.where(keep_top, 0, h)
    send_off = h - keep_off
    qa = jnp.where(my <= 1, 0, q)
    qb = q - qa

    def xfer(src, dst, stage, dev):
        rdma = pltpu.make_async_remote_copy(
            src_ref=src, dst_ref=dst,
            send_sem=send_sems.at[stage], recv_sem=recv_sems.at[stage],
            device_id=(dev,), device_id_type=pl.DeviceIdType.MESH,
        )
        rdma.start()
        rdma.wait()

    xfer(p_ref.at[pl.ds(send_off, h)], r1, 0, p1)
    acch[...] = (
        p_ref[pl.ds(keep_off, h), :].astype(jnp.float32)
        + r1[...].astype(jnp.float32)
    )

    s2[...] = acch[pl.ds(qb, q), :].astype(jnp.bfloat16)
    xfer(s2, r2, 1, p2)
    accq[...] = acch[pl.ds(qa, q), :] + r2[...].astype(jnp.float32)

    s3[...] = accq[...].astype(jnp.bfloat16)
    xfer(s3, r3, 2, p2)

    s4[pl.ds(qa, q), :] = s3[...]
    s4[pl.ds(qb, q), :] = r3[...]
    xfer(s4, r4, 3, p1)

    out_ref[pl.ds(keep_off + qa, q), :] = accq[...]
    out_ref[pl.ds(keep_off + qb, q), :] = r3[...].astype(jnp.float32)
    out_ref[pl.ds(send_off, h), :] = r4[...].astype(jnp.float32)

    @functools.partial(pl.run_scoped, sem=pltpu.SemaphoreType.REGULAR)
    def _(sem):
        for nbr in [p1, p2]:
            pl.semaphore_signal(
                sem, inc=1,
                device_id=(nbr,), device_id_type=pl.DeviceIdType.MESH,
            )
        pl.semaphore_wait(sem, 2)


def _ring_allreduce(partial):
    s, d = partial.shape
    h, q = s // 2, s // 4
    bf = jnp.bfloat16
    return pl.pallas_call(
        _allreduce_body,
        out_shape=jax.ShapeDtypeStruct((s, d), jnp.float32),
        in_specs=[pl.BlockSpec(memory_space=pltpu.VMEM)],
        out_specs=pl.BlockSpec(memory_space=pltpu.VMEM),
        scratch_shapes=[
            pltpu.VMEM((h, d), bf),
            pltpu.VMEM((q, d), bf),
            pltpu.VMEM((q, d), bf),
            pltpu.VMEM((q, d), bf),
            pltpu.VMEM((q, d), bf),
            pltpu.VMEM((h, d), bf),
            pltpu.VMEM((h, d), bf),
            pltpu.VMEM((h, d), jnp.float32),
            pltpu.VMEM((q, d), jnp.float32),
            pltpu.SemaphoreType.DMA((4,)),
            pltpu.SemaphoreType.DMA((4,)),
        ],
        compiler_params=pltpu.CompilerParams(collective_id=0),
    )(partial)


def kernel(x, Wq, K_ext, V_ext, Wo):
    i = lax.axis_index("i")
    sq = x.shape[1]
    skv = K_ext.shape[1]
    hq_local = Wq.shape[1] // DH

    xb = x[0].astype(jnp.bfloat16)
    q = (xb @ Wq.astype(jnp.bfloat16)).reshape(sq, hq_local, DH)
    k = lax.dynamic_slice_in_dim(K_ext[0], i * hq_local, hq_local, axis=1)
    v = lax.dynamic_slice_in_dim(V_ext[0], i * hq_local, hq_local, axis=1)

    scores = jnp.einsum(
        "ihd,jhd->hij", q, k.astype(jnp.bfloat16),
        preferred_element_type=jnp.float32,
    ) * SCALE
    qi = lax.broadcasted_iota(jnp.int32, (sq, skv), 0)
    ki = lax.broadcasted_iota(jnp.int32, (sq, skv), 1)
    mask = (jnp.abs(qi - ki) <= LOCAL_WINDOW) | (ki < GLOBAL_K) | (qi < GLOBAL_K)
    scores = jnp.where(mask[None, :, :], scores, -1e9)
    w = jax.nn.softmax(scores, axis=-1)

    ctx = jnp.einsum(
        "hij,jhd->ihd", w.astype(jnp.bfloat16), v.astype(jnp.bfloat16),
        preferred_element_type=jnp.float32,
    ).reshape(sq, hq_local * DH)
    partial = (ctx.astype(jnp.bfloat16) @ Wo.astype(jnp.bfloat16)).astype(
        jnp.bfloat16
    )

    out = _ring_allreduce(partial)
    return out.reshape(1, sq, Wo.shape[1])


# device time: 81076 ns/iter; 1.5971x vs baseline; 1.1936x over previous
import functools

import jax
import jax.numpy as jnp
from jax import lax
from jax.experimental import pallas as pl
from jax.experimental.pallas import tpu as pltpu

N_DEV = 4
SCALE = 0.08838834764831843
DH = 128
LOCAL_WINDOW = 128
GLOBAL_K = 32


def _fused_body(
    x_ref, wq_ref, k_ref, v_ref, wo_ref, out_ref,
    p, r1, s2, r2, s3, r3, s4, r4, acch, accq,
    send_sems, recv_sems,
):
    my = lax.axis_index("i")
    p1 = my ^ 1
    p2 = 3 - my

    sq, d = out_ref.shape
    h, q = sq // 2, sq // 4
    hq_local = k_ref.shape[0]

    qm = jnp.dot(
        x_ref[...], wq_ref[...], preferred_element_type=jnp.float32
    ).astype(jnp.bfloat16)

    qi = lax.broadcasted_iota(jnp.int32, (sq, sq), 0)
    ki = lax.broadcasted_iota(jnp.int32, (sq, sq), 1)
    mask = (
        (jnp.abs(qi - ki) <= LOCAL_WINDOW) | (ki < GLOBAL_K) | (qi < GLOBAL_K)
    )
    bias = jnp.where(mask, jnp.float32(0.0), jnp.float32(-1e9))

    ctx_parts = []
    for hh in range(hq_local):
        qh = qm[:, hh * DH:(hh + 1) * DH]
        s = lax.dot_general(
            qh, k_ref[hh], (((1,), (1,)), ((), ())),
            preferred_element_type=jnp.float32,
        ) * SCALE + bias
        m = jnp.max(s, axis=1, keepdims=True)
        e = jnp.exp(s - m)
        den = jnp.sum(e, axis=1, keepdims=True)
        w = (e / den).astype(jnp.bfloat16)
        ctx_parts.append(
            jnp.dot(w, v_ref[hh], preferred_element_type=jnp.float32).astype(
                jnp.bfloat16
            )
        )
    ctx = jnp.concatenate(ctx_parts, axis=1)
    p[...] = jnp.dot(
        ctx, wo_ref[...], preferred_element_type=jnp.float32
    ).astype(jnp.bfloat16)

    barrier_sem = pltpu.get_barrier_semaphore()
    for nbr in [p1, p2]:
        pl.semaphore_signal(
            barrier_sem, inc=1,
            device_id=(nbr,), device_id_type=pl.DeviceIdType.MESH,
        )
    pl.semaphore_wait(barrier_sem, 2)

    keep_top = (my == 0) | (my == 3)
    keep_off = jnp.where(keep_top, 0, h)
    send_off = h - keep_off
    qa = jnp.where(my <= 1, 0, q)
    qb = q - qa

    def xfer(src, dst, stage, dev):
        rdma = pltpu.make_async_remote_copy(
            src_ref=src, dst_ref=dst,
            send_sem=send_sems.at[stage], recv_sem=recv_sems.at[stage],
            device_id=(dev,), device_id_type=pl.DeviceIdType.MESH,
        )
        rdma.start()
        rdma.wait()

    xfer(p.at[pl.ds(send_off, h)], r1, 0, p1)
    acch[...] = (
        p[pl.ds(keep_off, h), :].astype(jnp.float32)
        + r1[...].astype(jnp.float32)
    )

    s2[...] = acch[pl.ds(qb, q), :].astype(jnp.bfloat16)
    xfer(s2, r2, 1, p2)
    accq[...] = acch[pl.ds(qa, q), :] + r2[...].astype(jnp.float32)

    s3[...] = accq[...].astype(jnp.bfloat16)
    xfer(s3, r3, 2, p2)

    s4[pl.ds(qa, q), :] = s3[...]
    s4[pl.ds(qb, q), :] = r3[...]
    xfer(s4, r4, 3, p1)

    out_ref[pl.ds(keep_off + qa, q), :] = accq[...]
    out_ref[pl.ds(keep_off + qb, q), :] = r3[...].astype(jnp.float32)
    out_ref[pl.ds(send_off, h), :] = r4[...].astype(jnp.float32)

    @functools.partial(pl.run_scoped, sem=pltpu.SemaphoreType.REGULAR)
    def _(sem):
        for nbr in [p1, p2]:
            pl.semaphore_signal(
                sem, inc=1,
                device_id=(nbr,), device_id_type=pl.DeviceIdType.MESH,
            )
        pl.semaphore_wait(sem, 2)


def kernel(x, Wq, K_ext, V_ext, Wo):
    i = lax.axis_index("i")
    sq = x.shape[1]
    d = Wo.shape[1]
    hq_local = Wq.shape[1] // DH
    h, q = sq // 2, sq // 4
    bf = jnp.bfloat16

    xb = x[0].astype(bf)
    k = jnp.swapaxes(
        lax.dynamic_slice_in_dim(K_ext[0], i * hq_local, hq_local, axis=1),
        0, 1,
    ).astype(bf)
    v = jnp.swapaxes(
        lax.dynamic_slice_in_dim(V_ext[0], i * hq_local, hq_local, axis=1),
        0, 1,
    ).astype(bf)

    out = pl.pallas_call(
        _fused_body,
        out_shape=jax.ShapeDtypeStruct((sq, d), jnp.float32),
        in_specs=[pl.BlockSpec(memory_space=pltpu.VMEM)] * 5,
        out_specs=pl.BlockSpec(memory_space=pltpu.VMEM),
        scratch_shapes=[
            pltpu.VMEM((sq, d), bf),
            pltpu.VMEM((h, d), bf),
            pltpu.VMEM((q, d), bf),
            pltpu.VMEM((q, d), bf),
            pltpu.VMEM((q, d), bf),
            pltpu.VMEM((q, d), bf),
            pltpu.VMEM((h, d), bf),
            pltpu.VMEM((h, d), bf),
            pltpu.VMEM((h, d), jnp.float32),
            pltpu.VMEM((q, d), jnp.float32),
            pltpu.SemaphoreType.DMA((4,)),
            pltpu.SemaphoreType.DMA((4,)),
        ],
        compiler_params=pltpu.CompilerParams(collective_id=0),
    )(xb, Wq.astype(bf), k, v, Wo.astype(bf))
    return out.reshape(1, sq, d)


# device time: 64357 ns/iter; 2.0120x vs baseline; 1.2598x over previous
import functools

import jax
import jax.numpy as jnp
from jax import lax
from jax.experimental import pallas as pl
from jax.experimental.pallas import tpu as pltpu

N_DEV = 4
SCALE = 0.08838834764831843
DH = 128
LOCAL_WINDOW = 128
GLOBAL_K = 32


def _fused_body(
    x_ref, wq_ref, k_ref, v_ref, wo_ref, out_ref,
    p, r1, s2, r2, s3, r3, s4, r4, acch, accq,
    send_sems, recv_sems,
):
    my = lax.axis_index("i")
    p1 = my ^ 1
    p2 = 3 - my

    sq, d = out_ref.shape
    h, q = sq // 2, sq // 4
    hq_local = k_ref.shape[0]

    qm = jnp.dot(
        x_ref[...], wq_ref[...], preferred_element_type=jnp.float32
    ).astype(jnp.bfloat16)

    qi = lax.broadcasted_iota(jnp.int32, (sq, sq), 0)
    ki = lax.broadcasted_iota(jnp.int32, (sq, sq), 1)
    mask = (
        (jnp.abs(qi - ki) <= LOCAL_WINDOW) | (ki < GLOBAL_K) | (qi < GLOBAL_K)
    )
    bias = jnp.where(mask, jnp.float32(0.0), jnp.float32(-1e9))

    ctx_parts = []
    for hh in range(hq_local):
        qh = qm[:, hh * DH:(hh + 1) * DH]
        s = lax.dot_general(
            qh, k_ref[hh], (((1,), (1,)), ((), ())),
            preferred_element_type=jnp.float32,
        ) * SCALE + bias
        m = jnp.max(s, axis=1, keepdims=True)
        e = jnp.exp(s - m)
        den = jnp.sum(e, axis=1, keepdims=True)
        w = (e / den).astype(jnp.bfloat16)
        ctx_parts.append(
            jnp.dot(w, v_ref[hh], preferred_element_type=jnp.float32).astype(
                jnp.bfloat16
            )
        )
    ctx = jnp.concatenate(ctx_parts, axis=1)
    p[...] = jnp.dot(
        ctx, wo_ref[...], preferred_element_type=jnp.float32
    ).astype(jnp.bfloat16)

    barrier_sem = pltpu.get_barrier_semaphore()
    for nbr in [p1, p2]:
        pl.semaphore_signal(
            barrier_sem, inc=1,
            device_id=(nbr,), device_id_type=pl.DeviceIdType.MESH,
        )
    pl.semaphore_wait(barrier_sem, 2)

    cw = d // 2
    cols = [pl.ds(0, cw), pl.ds(cw, cw)]
    part = [[p1, p2, p2, p1], [p2, p1, p1, p2]]
    keep_top = [(my == 0) | (my == 3), my <= 1]
    keep_off = [jnp.where(kt, 0, h) for kt in keep_top]
    send_off = [h - ko for ko in keep_off]
    qa = [
        jnp.where(my <= 1, 0, q),
        jnp.where((my == 0) | (my == 2), 0, q),
    ]
    qb = [q - x for x in qa]

    def xfer(stage, srcs, dsts):
        rdmas = []
        for s in (0, 1):
            rdma = pltpu.make_async_remote_copy(
                src_ref=srcs[s], dst_ref=dsts[s],
                send_sem=send_sems.at[stage * 2 + s],
                recv_sem=recv_sems.at[stage * 2 + s],
                device_id=(part[s][stage],),
                device_id_type=pl.DeviceIdType.MESH,
            )
            rdma.start()
            rdmas.append(rdma)
        for rdma in rdmas:
            rdma.wait()

    xfer(
        0,
        [p.at[pl.ds(send_off[s], h), cols[s]] for s in (0, 1)],
        [r1.at[:, cols[s]] for s in (0, 1)],
    )
    for s in (0, 1):
        acch[:, cols[s]] = (
            p[pl.ds(keep_off[s], h), cols[s]].astype(jnp.float32)
            + r1[:, cols[s]].astype(jnp.float32)
        )

    for s in (0, 1):
        s2[:, cols[s]] = acch[pl.ds(qb[s], q), cols[s]].astype(jnp.bfloat16)
    xfer(
        1,
        [s2.at[:, cols[s]] for s in (0, 1)],
        [r2.at[:, cols[s]] for s in (0, 1)],
    )
    for s in (0, 1):
        accq[:, cols[s]] = (
            acch[pl.ds(qa[s], q), cols[s]]
            + r2[:, cols[s]].astype(jnp.float32)
        )

    s3[...] = accq[...].astype(jnp.bfloat16)
    xfer(
        2,
        [s3.at[:, cols[s]] for s in (0, 1)],
        [r3.at[:, cols[s]] for s in (0, 1)],
    )

    for s in (0, 1):
        s4[pl.ds(qa[s], q), cols[s]] = s3[:, cols[s]]
        s4[pl.ds(qb[s], q), cols[s]] = r3[:, cols[s]]
    xfer(
        3,
        [s4.at[:, cols[s]] for s in (0, 1)],
        [r4.at[:, cols[s]] for s in (0, 1)],
    )

    for s in (0, 1):
        out_ref[pl.ds(keep_off[s] + qa[s], q), cols[s]] = accq[:, cols[s]]
        out_ref[pl.ds(keep_off[s] + qb[s], q), cols[s]] = r3[
            :, cols[s]
        ].astype(jnp.float32)
        out_ref[pl.ds(send_off[s], h), cols[s]] = r4[:, cols[s]].astype(
            jnp.float32
        )

    @functools.partial(pl.run_scoped, sem=pltpu.SemaphoreType.REGULAR)
    def _(sem):
        for nbr in [p1, p2]:
            pl.semaphore_signal(
                sem, inc=1,
                device_id=(nbr,), device_id_type=pl.DeviceIdType.MESH,
            )
        pl.semaphore_wait(sem, 2)


def kernel(x, Wq, K_ext, V_ext, Wo):
    i = lax.axis_index("i")
    sq = x.shape[1]
    d = Wo.shape[1]
    hq_local = Wq.shape[1] // DH
    h, q = sq // 2, sq // 4
    bf = jnp.bfloat16

    xb = x[0].astype(bf)
    k = jnp.swapaxes(
        lax.dynamic_slice_in_dim(K_ext[0], i * hq_local, hq_local, axis=1),
        0, 1,
    ).astype(bf)
    v = jnp.swapaxes(
        lax.dynamic_slice_in_dim(V_ext[0], i * hq_local, hq_local, axis=1),
        0, 1,
    ).astype(bf)

    out = pl.pallas_call(
        _fused_body,
        out_shape=jax.ShapeDtypeStruct((sq, d), jnp.float32),
        in_specs=[pl.BlockSpec(memory_space=pltpu.VMEM)] * 5,
        out_specs=pl.BlockSpec(memory_space=pltpu.VMEM),
        scratch_shapes=[
            pltpu.VMEM((sq, d), bf),
            pltpu.VMEM((h, d), bf),
            pltpu.VMEM((q, d), bf),
            pltpu.VMEM((q, d), bf),
            pltpu.VMEM((q, d), bf),
            pltpu.VMEM((q, d), bf),
            pltpu.VMEM((h, d), bf),
            pltpu.VMEM((h, d), bf),
            pltpu.VMEM((h, d), jnp.float32),
            pltpu.VMEM((q, d), jnp.float32),
            pltpu.SemaphoreType.DMA((8,)),
            pltpu.SemaphoreType.DMA((8,)),
        ],
        compiler_params=pltpu.CompilerParams(collective_id=0),
    )(xb, Wq.astype(bf), k, v, Wo.astype(bf))
    return out.reshape(1, sq, d)
